# Initial kernel scaffold; baseline (speedup 1.0000x reference)
import jax
import jax.numpy as jnp
from jax.experimental import pallas as pl


def kernel(x, W_enc, b_enc, W_dec, b_dec):
    raise NotImplementedError("write your pallas kernel here")


# speedup vs baseline: 2.1114x; 2.1114x over previous
import functools

import jax
import jax.numpy as jnp
from jax import lax
from jax.experimental import pallas as pl
from jax.experimental.pallas import tpu as pltpu
from jax.experimental.pallas import tpu_sc as plsc

D_IN_ = 768
HID_ = 24576
BATCH_ = 8192
K_ = 32
CHUNK_ = 128
NCHUNK_ = HID_ // CHUNK_

BT_ = 256
HT_ = 2048
NBT_ = BATCH_ // BT_
NHT_ = HID_ // HT_
CPT_ = HT_ // CHUNK_

NT_ = 64


def _tgrid(i_f32):
    return jnp.where(i_f32 < 16.0, -4.0 + 0.4 * i_f32,
                     2.4 + 0.03 * (i_f32 - 16.0))

NW_ = 32
RPW_ = BATCH_ // NW_
GB_ = 64
CAP_ = 512
CBUF_ = CAP_ + CHUNK_ + 16


def _enc_body(x_ref, we_ref, be_ref, bd_ref, pre_ref, cmax_ref, cnt_ref,
              cacc_ref):
    h = pl.program_id(0)
    b = pl.program_id(1)
    xt = x_ref[...] - bd_ref[...]
    p = lax.dot_general(
        xt, we_ref[...], (((1,), (1,)), ((), ())),
        preferred_element_type=jnp.float32,
        precision=lax.Precision.DEFAULT,
    ) + be_ref[...]
    pre_ref[...] = p
    cm = jnp.max(p.reshape(BT_, CPT_, CHUNK_), axis=2)
    cmax_ref[...] = cm.reshape(1, BT_, CPT_)

    tgrid = _tgrid(
        lax.broadcasted_iota(jnp.int32, (1, NT_), 1).astype(jnp.float32))
    cnt = jnp.zeros((BT_, NT_), dtype=jnp.float32)
    for c in range(CPT_):
        cnt = cnt + (cm[:, c:c + 1] >= tgrid).astype(jnp.float32)

    @pl.when(h == 0)
    def _():
        cacc_ref[b] = cnt

    @pl.when(h != 0)
    def _():
        cacc_ref[b] = cacc_ref[b] + cnt

    cnt_ref[...] = cacc_ref[b]


def _encode(x, W_enc, b_enc, b_dec):
    return pl.pallas_call(
        _enc_body,
        grid=(NHT_, NBT_),
        in_specs=[
            pl.BlockSpec((BT_, D_IN_), lambda h, b: (b, 0)),
            pl.BlockSpec((HT_, D_IN_), lambda h, b: (h, 0)),
            pl.BlockSpec((1, HT_), lambda h, b: (0, h)),
            pl.BlockSpec((1, D_IN_), lambda h, b: (0, 0)),
        ],
        out_specs=[
            pl.BlockSpec((BT_, HT_), lambda h, b: (b, h)),
            pl.BlockSpec((1, BT_, CPT_), lambda h, b: (h, b, 0)),
            pl.BlockSpec((BT_, NT_), lambda h, b: (b, 0)),
        ],
        out_shape=[
            jax.ShapeDtypeStruct((BATCH_, HID_), jnp.float32),
            jax.ShapeDtypeStruct((NHT_, BATCH_, CPT_), jnp.float32),
            jax.ShapeDtypeStruct((BATCH_, NT_), jnp.float32),
        ],
        scratch_shapes=[pltpu.VMEM((NBT_, BT_, NT_), jnp.float32)],
    )(x, W_enc, b_enc.reshape(1, HID_), b_dec.reshape(1, D_IN_))


_NEG = -3.0e38
_POS = 3.0e38


def _sc_body(pre2, cmaxh, cntsh, wdech, bdech, outh,
             counts_v, cmax_v, idb, chkb, cu, ci, selu, seli, selv,
             wrows, acc, bdv, tmp16, sem1, sem2):
    wid = lax.axis_index("s") * 2 + lax.axis_index("c")
    row0 = wid * RPW_
    pltpu.sync_copy(cntsh.at[pl.ds(row0, RPW_)], counts_v)
    pltpu.sync_copy(bdech, bdv)
    iota = lax.iota(jnp.int32, 16)
    iota_f = iota.astype(jnp.float32)
    tvecs = [_tgrid(iota_f + float(q * 16)) for q in range(NT_ // 16)]
    k32 = jnp.full((16,), float(K_), dtype=jnp.float32)

    def row_body(rl, _carry):
        row = row0 + rl
        pltpu.sync_copy(cmaxh.at[row], cmax_v)

        th_acc = jnp.full((16,), _NEG, dtype=jnp.float32)
        for q in range(NT_ // 16):
            c = counts_v[rl, pl.ds(q * 16, 16)]
            th_acc = jnp.maximum(th_acc,
                                 jnp.where(c >= k32, tvecs[q], _NEG))
        m = th_acc
        for sh in (8, 4, 2, 1):
            tmp16[...] = m
            m = jnp.maximum(m, plsc.load_gather(tmp16, [iota ^ sh]))
        th = m[0]
        thv = jnp.full((16,), th, dtype=jnp.float32)

        base = row * NCHUNK_
        basev = jnp.full((16,), base, dtype=jnp.int32)
        for qv in range(7 * GB_ // 16 + 1):
            idb[pl.ds(qv * 16, 16)] = basev

        def cscan(q, ptr):
            v = cmax_v[pl.ds(q * 16, 16)]
            m = v >= thv
            ids = jnp.full((16,), base + q * 16, dtype=jnp.int32) + iota
            plsc.store_compressed(idb.at[pl.ds(ptr, 16)], ids, mask=m)
            return ptr + plsc.all_reduce_population_count(m)[0]

        nc = lax.fori_loop(0, NCHUNK_ // 16, cscan, 0)
        nb = (nc + GB_ - 1) // GB_

        one = jnp.full((16,), 1, dtype=jnp.int32)
        zero = jnp.zeros((16,), dtype=jnp.int32)

        def select32(cnt):
            cu[pl.ds(cnt, 16)] = jnp.zeros((16,), dtype=jnp.uint32)
            nv = (cnt + 15) // 16

            def cnt_ge(t_u32):
                ts = jnp.full((16,), 0, dtype=jnp.uint32) + t_u32

                def cb(q, a):
                    u = cu[pl.ds(q * 16, 16)]
                    return a + plsc.all_reduce_population_count(u >= ts)

                return lax.fori_loop(0, nv, cb, zero)[0]

            def bis(_, lohi):
                lo, hi = lohi
                mid = lo + (hi - lo) // jnp.uint32(2)
                big = cnt_ge(mid) >= K_
                return (jnp.where(big, mid, lo), jnp.where(big, hi, mid))

            lo, _hi = lax.fori_loop(
                0, 32, bis,
                (jnp.uint32(0), jnp.uint32(0xFFFFFFFF)))
            tsel = jnp.full((16,), 0, dtype=jnp.uint32) + lo
            n_gt = cnt_ge(lo + jnp.uint32(1))

            def ext(q, carry):
                p, neq = carry
                u = cu[pl.ds(q * 16, 16)]
                gi = ci[pl.ds(q * 16, 16)]
                m_gt = u > tsel
                m_eq = u == tsel

                def drop(m):
                    mi = lax.rev(jnp.where(m, one, zero), (0,))
                    last = 15 - plsc.all_reduce_ffs(mi == one)[0]
                    return m & ~(iota == (zero + last))

                m_eq = lax.while_loop(
                    lambda m: plsc.all_reduce_population_count(m)[0] > neq,
                    drop, m_eq)
                m = m_gt | m_eq
                plsc.store_compressed(selu.at[pl.ds(p, 16)], u, mask=m)
                plsc.store_compressed(seli.at[pl.ds(p, 16)], gi, mask=m)
                return (p + plsc.all_reduce_population_count(m)[0],
                        neq - plsc.all_reduce_population_count(m_eq)[0])

            lax.fori_loop(0, nv, ext, (0, K_ - n_gt))

        def batch_body(bb, ptr):
            pltpu.async_copy(pre2.at[idb.at[pl.ds(bb * GB_, GB_)]],
                             chkb, sem1).wait()

            def chunk_body(i, ptr2):
                slot = bb * GB_ + i
                cid = idb[pl.ds(slot, 16)][0]
                over = ptr2 > CAP_

                @pl.when(over)
                def _():
                    select32(ptr2)
                    for t2 in range(K_ // 16):
                        cu[pl.ds(t2 * 16, 16)] = selu[pl.ds(t2 * 16, 16)]
                        ci[pl.ds(t2 * 16, 16)] = seli[pl.ds(t2 * 16, 16)]

                ptr2 = jnp.where(over, K_, ptr2)
                th_eff = jnp.where(slot < nc, th, _POS)
                te = jnp.full((16,), th_eff, dtype=jnp.float32)
                jb = (cid - base) * CHUNK_
                for qq in range(CHUNK_ // 16):
                    v = chkb[i, pl.ds(qq * 16, 16)]
                    m = v >= te
                    s32 = plsc.bitcast(v, jnp.int32)
                    u = plsc.bitcast(
                        jnp.where(s32 < 0, ~s32,
                                  s32 | jnp.int32(-2147483648)),
                        jnp.uint32)
                    gi = jnp.full((16,), jb + qq * 16, dtype=jnp.int32) + iota
                    plsc.store_compressed(cu.at[pl.ds(ptr2, 16)], u, mask=m)
                    plsc.store_compressed(ci.at[pl.ds(ptr2, 16)], gi, mask=m)
                    ptr2 = ptr2 + plsc.all_reduce_population_count(m)[0]
                return ptr2

            return lax.fori_loop(0, GB_, chunk_body, ptr)

        cnt = lax.fori_loop(0, nb, batch_body, 0)
        select32(cnt)

        for t2 in range(K_ // 16):
            uu = selu[pl.ds(t2 * 16, 16)]
            s32 = plsc.bitcast(uu, jnp.int32)
            selv[pl.ds(t2 * 16, 16)] = plsc.bitcast(
                jnp.where(s32 < 0, s32 & jnp.int32(0x7FFFFFFF), ~s32),
                jnp.float32)

        pltpu.async_copy(wdech.at[seli.at[pl.ds(0, K_)]], wrows, sem2).wait()
        for j in range(D_IN_ // 16):
            acc[pl.ds(j * 16, 16)] = bdv[pl.ds(j * 16, 16)]

        def dk(k, _):
            vs = plsc.load_gather(selv, [jnp.full((16,), 0, jnp.int32) + k])
            for j in range(D_IN_ // 16):
                w = wrows[k, pl.ds(j * 16, 16)]
                plsc.addupdate(acc.at[pl.ds(j * 16, 16)], vs * w)
            return 0

        lax.fori_loop(0, K_, dk, 0)
        pltpu.sync_copy(acc, outh.at[row])
        return 0

    lax.fori_loop(0, RPW_, row_body, 0)


def _sc_decode(pre, cmax, counts, W_dec, b_dec):
    pre2 = pre.reshape(BATCH_ * NCHUNK_, CHUNK_)
    mesh = plsc.VectorSubcoreMesh(core_axis_name="c", subcore_axis_name="s",
                                  num_cores=2, num_subcores=16)
    f = pl.kernel(
        _sc_body,
        out_type=jax.ShapeDtypeStruct((BATCH_, D_IN_), jnp.float32),
        mesh=mesh,
        compiler_params=pltpu.CompilerParams(needs_layout_passes=False),
        scratch_types=[
            pltpu.VMEM((RPW_, NT_), jnp.float32),
            pltpu.VMEM((NCHUNK_,), jnp.float32),
            pltpu.VMEM((7 * GB_ + 16,), jnp.int32),
            pltpu.VMEM((GB_, CHUNK_), jnp.float32),
            pltpu.VMEM((CBUF_,), jnp.uint32),
            pltpu.VMEM((CBUF_,), jnp.int32),
            pltpu.VMEM((K_ + 16,), jnp.uint32),
            pltpu.VMEM((K_ + 16,), jnp.int32),
            pltpu.VMEM((K_,), jnp.float32),
            pltpu.VMEM((K_, D_IN_), jnp.float32),
            pltpu.VMEM((D_IN_,), jnp.float32),
            pltpu.VMEM((D_IN_,), jnp.float32),
            pltpu.VMEM((16,), jnp.float32),
            pltpu.SemaphoreType.DMA,
            pltpu.SemaphoreType.DMA,
        ],
    )
    return f(pre2, cmax, counts, W_dec, b_dec)


@jax.jit
def _run(x, W_enc, b_enc, W_dec, b_dec):
    pre, cmax3, counts = _encode(x, W_enc, b_enc, b_dec)
    cmax = cmax3.transpose(1, 0, 2).reshape(BATCH_, NCHUNK_)
    return _sc_decode(pre, cmax, counts, W_dec, b_dec)


def kernel(x, W_enc, b_enc, W_dec, b_dec):
    return _run(x, W_enc, b_enc, W_dec, b_dec)


# speedup vs baseline: 4.0396x; 1.9132x over previous
import functools

import jax
import jax.numpy as jnp
from jax import lax
from jax.experimental import pallas as pl
from jax.experimental.pallas import tpu as pltpu
from jax.experimental.pallas import tpu_sc as plsc

D_IN_ = 768
HID_ = 24576
BATCH_ = 8192
K_ = 32
CHUNK_ = 128
NCHUNK_ = HID_ // CHUNK_

BT_ = 256
HT_ = 2048
NBT_ = BATCH_ // BT_
NHT_ = HID_ // HT_
CPT_ = HT_ // CHUNK_

NT_ = 64


def _tgrid(i_f32):
    return jnp.where(i_f32 < 16.0, -4.0 + 0.4 * i_f32,
                     2.4 + 0.03 * (i_f32 - 16.0))

NW_ = 32
RPW_ = BATCH_ // NW_
GB_ = 64
CAP_ = 512
CBUF_ = CAP_ + CHUNK_ + 16


def _enc_body(x_ref, we_ref, be_ref, bd_ref, pre_ref, cmax_ref):
    xt = x_ref[...] - bd_ref[...]
    p = lax.dot_general(
        xt, we_ref[...], (((1,), (1,)), ((), ())),
        preferred_element_type=jnp.float32,
        precision=lax.Precision.DEFAULT,
    ) + be_ref[...]
    pre_ref[...] = p
    cm = jnp.max(p.reshape(BT_, CPT_, CHUNK_), axis=2)
    cmax_ref[...] = cm.reshape(1, BT_ // 8, 8 * CPT_)


def _encode(x, W_enc, b_enc, b_dec):
    return pl.pallas_call(
        _enc_body,
        grid=(NHT_, NBT_),
        in_specs=[
            pl.BlockSpec((BT_, D_IN_), lambda h, b: (b, 0)),
            pl.BlockSpec((HT_, D_IN_), lambda h, b: (h, 0)),
            pl.BlockSpec((1, HT_), lambda h, b: (0, h)),
            pl.BlockSpec((1, D_IN_), lambda h, b: (0, 0)),
        ],
        out_specs=[
            pl.BlockSpec((BT_, HT_), lambda h, b: (b, h)),
            pl.BlockSpec((1, BT_ // 8, 8 * CPT_), lambda h, b: (h, b, 0)),
        ],
        out_shape=[
            jax.ShapeDtypeStruct((BATCH_, HID_), jnp.float32),
            jax.ShapeDtypeStruct((NHT_, BATCH_ // 8, 8 * CPT_), jnp.float32),
        ],
    )(x, W_enc, b_enc.reshape(1, HID_), b_dec.reshape(1, D_IN_))


_NEG = -3.0e38
_POS = 3.0e38


def _sc_body(pre2, cmax3h, wdech, bdech, outh,
             cm3_v, cmax_v, idb, chkb, cu, ci, selu, seli, selv,
             wrows, acc, bdv, tmp16, sem1, sem2):
    wid = lax.axis_index("s") * 2 + lax.axis_index("c")
    row0 = wid * RPW_
    rg0 = pl.multiple_of(row0 // 8, RPW_ // 8)
    for h0 in range(NHT_):
        pltpu.sync_copy(cmax3h.at[h0, pl.ds(rg0, RPW_ // 8), :],
                        cm3_v.at[h0])
    pltpu.sync_copy(bdech, bdv)
    iota = lax.iota(jnp.int32, 16)

    def row_body(rl, _carry):
        row = row0 + rl
        vmx = jnp.full((16,), _NEG, dtype=jnp.float32)
        vmn = jnp.full((16,), _POS, dtype=jnp.float32)
        rg = rl // 8
        off = (rl % 8) * 16
        for h in range(NHT_):
            v = cm3_v[h, rg, pl.ds(off, 16)]
            cmax_v[pl.ds(h * 16, 16)] = v
            vmx = jnp.maximum(vmx, v)
            vmn = jnp.minimum(vmn, v)
        for sh in (8, 4, 2, 1):
            tmp16[...] = vmx
            vmx = jnp.maximum(vmx, plsc.load_gather(tmp16, [iota ^ sh]))
            tmp16[...] = vmn
            vmn = jnp.minimum(vmn, plsc.load_gather(tmp16, [iota ^ sh]))
        rmax = vmx[0]
        rmin = vmn[0]

        def th_cnt(t):
            ts = jnp.full((16,), 0.0, dtype=jnp.float32) + t
            a = jnp.zeros((16,), dtype=jnp.int32)
            for h in range(NHT_):
                a = a + plsc.all_reduce_population_count(
                    cmax_v[pl.ds(h * 16, 16)] >= ts)
            return a[0]

        def bisf(_, lohi):
            lo, hi = lohi
            mid = 0.5 * (lo + hi)
            big = th_cnt(mid) >= K_
            return (jnp.where(big, mid, lo), jnp.where(big, hi, mid))

        hi0 = rmax + (jnp.abs(rmax) * 1e-6 + 1e-30)
        th, _unused = lax.fori_loop(0, 20, bisf, (rmin, hi0))
        thv = jnp.full((16,), th, dtype=jnp.float32)

        base = row * NCHUNK_
        basev = jnp.full((16,), base, dtype=jnp.int32)
        for qv in range(7 * GB_ // 16 + 1):
            idb[pl.ds(qv * 16, 16)] = basev

        def cscan(q, ptr):
            v = cmax_v[pl.ds(q * 16, 16)]
            m = v >= thv
            ids = jnp.full((16,), base + q * 16, dtype=jnp.int32) + iota
            plsc.store_compressed(idb.at[pl.ds(ptr, 16)], ids, mask=m)
            return ptr + plsc.all_reduce_population_count(m)[0]

        nc = lax.fori_loop(0, NCHUNK_ // 16, cscan, 0)
        nb = (nc + GB_ - 1) // GB_

        one = jnp.full((16,), 1, dtype=jnp.int32)
        zero = jnp.zeros((16,), dtype=jnp.int32)

        def select32(cnt):
            cu[pl.ds(cnt, 16)] = jnp.zeros((16,), dtype=jnp.uint32)
            nv = (cnt + 15) // 16

            def cnt_ge(t_u32):
                ts = jnp.full((16,), 0, dtype=jnp.uint32) + t_u32

                def cb(q, a):
                    u = cu[pl.ds(q * 16, 16)]
                    return a + plsc.all_reduce_population_count(u >= ts)

                return lax.fori_loop(0, nv, cb, zero)[0]

            def bis(_, lohi):
                lo, hi = lohi
                mid = lo + (hi - lo) // jnp.uint32(2)
                big = cnt_ge(mid) >= K_
                return (jnp.where(big, mid, lo), jnp.where(big, hi, mid))

            lo, _hi = lax.fori_loop(
                0, 32, bis,
                (jnp.uint32(0), jnp.uint32(0xFFFFFFFF)))
            tsel = jnp.full((16,), 0, dtype=jnp.uint32) + lo
            n_gt = cnt_ge(lo + jnp.uint32(1))

            def ext(q, carry):
                p, neq = carry
                u = cu[pl.ds(q * 16, 16)]
                gi = ci[pl.ds(q * 16, 16)]
                m_gt = u > tsel
                m_eq = u == tsel

                def drop(m):
                    mi = lax.rev(jnp.where(m, one, zero), (0,))
                    last = 15 - plsc.all_reduce_ffs(mi == one)[0]
                    return m & ~(iota == (zero + last))

                m_eq = lax.while_loop(
                    lambda m: plsc.all_reduce_population_count(m)[0] > neq,
                    drop, m_eq)
                m = m_gt | m_eq
                plsc.store_compressed(selu.at[pl.ds(p, 16)], u, mask=m)
                plsc.store_compressed(seli.at[pl.ds(p, 16)], gi, mask=m)
                return (p + plsc.all_reduce_population_count(m)[0],
                        neq - plsc.all_reduce_population_count(m_eq)[0])

            lax.fori_loop(0, nv, ext, (0, K_ - n_gt))

        def batch_body(bb, ptr):
            pltpu.async_copy(pre2.at[idb.at[pl.ds(bb * GB_, GB_)]],
                             chkb, sem1).wait()

            def chunk_body(i, ptr2):
                slot = bb * GB_ + i
                cid = idb[pl.ds(slot, 16)][0]
                over = ptr2 > CAP_

                @pl.when(over)
                def _():
                    select32(ptr2)
                    for t2 in range(K_ // 16):
                        cu[pl.ds(t2 * 16, 16)] = selu[pl.ds(t2 * 16, 16)]
                        ci[pl.ds(t2 * 16, 16)] = seli[pl.ds(t2 * 16, 16)]

                ptr2 = jnp.where(over, K_, ptr2)
                th_eff = jnp.where(slot < nc, th, _POS)
                te = jnp.full((16,), th_eff, dtype=jnp.float32)
                jb = (cid - base) * CHUNK_
                for qq in range(CHUNK_ // 16):
                    v = chkb[i, pl.ds(qq * 16, 16)]
                    m = v >= te
                    s32 = plsc.bitcast(v, jnp.int32)
                    u = plsc.bitcast(
                        jnp.where(s32 < 0, ~s32,
                                  s32 | jnp.int32(-2147483648)),
                        jnp.uint32)
                    gi = jnp.full((16,), jb + qq * 16, dtype=jnp.int32) + iota
                    plsc.store_compressed(cu.at[pl.ds(ptr2, 16)], u, mask=m)
                    plsc.store_compressed(ci.at[pl.ds(ptr2, 16)], gi, mask=m)
                    ptr2 = ptr2 + plsc.all_reduce_population_count(m)[0]
                return ptr2

            return lax.fori_loop(0, GB_, chunk_body, ptr)

        cnt = lax.fori_loop(0, nb, batch_body, 0)
        select32(cnt)

        for t2 in range(K_ // 16):
            uu = selu[pl.ds(t2 * 16, 16)]
            s32 = plsc.bitcast(uu, jnp.int32)
            selv[pl.ds(t2 * 16, 16)] = plsc.bitcast(
                jnp.where(s32 < 0, s32 & jnp.int32(0x7FFFFFFF), ~s32),
                jnp.float32)

        pltpu.async_copy(wdech.at[seli.at[pl.ds(0, K_)]], wrows, sem2).wait()
        for j in range(D_IN_ // 16):
            acc[pl.ds(j * 16, 16)] = bdv[pl.ds(j * 16, 16)]

        def dk(k, _):
            vs = plsc.load_gather(selv, [jnp.full((16,), 0, jnp.int32) + k])
            for j in range(D_IN_ // 16):
                w = wrows[k, pl.ds(j * 16, 16)]
                plsc.addupdate(acc.at[pl.ds(j * 16, 16)], vs * w)
            return 0

        lax.fori_loop(0, K_, dk, 0)
        pltpu.sync_copy(acc, outh.at[row])
        return 0

    lax.fori_loop(0, RPW_, row_body, 0)


def _sc_decode(pre, cmax3, W_dec, b_dec):
    pre2 = pre.reshape(BATCH_ * NCHUNK_, CHUNK_)
    mesh = plsc.VectorSubcoreMesh(core_axis_name="c", subcore_axis_name="s",
                                  num_cores=2, num_subcores=16)
    f = pl.kernel(
        _sc_body,
        out_type=jax.ShapeDtypeStruct((BATCH_, D_IN_), jnp.float32),
        mesh=mesh,
        compiler_params=pltpu.CompilerParams(needs_layout_passes=False),
        scratch_types=[
            pltpu.VMEM((NHT_, RPW_ // 8, 128), jnp.float32),
            pltpu.VMEM((NCHUNK_,), jnp.float32),
            pltpu.VMEM((7 * GB_ + 16,), jnp.int32),
            pltpu.VMEM((GB_, CHUNK_), jnp.float32),
            pltpu.VMEM((CBUF_,), jnp.uint32),
            pltpu.VMEM((CBUF_,), jnp.int32),
            pltpu.VMEM((K_ + 16,), jnp.uint32),
            pltpu.VMEM((K_ + 16,), jnp.int32),
            pltpu.VMEM((K_,), jnp.float32),
            pltpu.VMEM((K_, D_IN_), jnp.float32),
            pltpu.VMEM((D_IN_,), jnp.float32),
            pltpu.VMEM((D_IN_,), jnp.float32),
            pltpu.VMEM((16,), jnp.float32),
            pltpu.SemaphoreType.DMA,
            pltpu.SemaphoreType.DMA,
        ],
    )
    return f(pre2, cmax3, W_dec, b_dec)


@jax.jit
def _run(x, W_enc, b_enc, W_dec, b_dec):
    pre, cmax3 = _encode(x, W_enc, b_enc, b_dec)
    return _sc_decode(pre, cmax3, W_dec, b_dec)


def kernel(x, W_enc, b_enc, W_dec, b_dec):
    return _run(x, W_enc, b_enc, W_dec, b_dec)


# speedup vs baseline: 5.1253x; 1.2688x over previous
import functools

import jax
import jax.numpy as jnp
from jax import lax
from jax.experimental import pallas as pl
from jax.experimental.pallas import tpu as pltpu
from jax.experimental.pallas import tpu_sc as plsc

D_IN_ = 768
HID_ = 24576
BATCH_ = 8192
K_ = 32
CHUNK_ = 128
NCHUNK_ = HID_ // CHUNK_

BT_ = 256
HT_ = 2048
NBT_ = BATCH_ // BT_
NHT_ = HID_ // HT_
CPT_ = HT_ // CHUNK_

NT_ = 64


def _tgrid(i_f32):
    return jnp.where(i_f32 < 16.0, -4.0 + 0.4 * i_f32,
                     2.4 + 0.03 * (i_f32 - 16.0))

NW_ = 32
RPW_ = BATCH_ // NW_
GB_ = 64
CAP_ = 512
CBUF_ = CAP_ + CHUNK_ + 16


def _enc_body(x_ref, we_ref, be_ref, bd_ref, pre_ref, cmax_ref):
    xt = x_ref[...] - bd_ref[...]
    p = lax.dot_general(
        xt, we_ref[...], (((1,), (1,)), ((), ())),
        preferred_element_type=jnp.float32,
        precision=lax.Precision.DEFAULT,
    ) + be_ref[...]
    pre_ref[...] = p.reshape(BT_ * CPT_, CHUNK_)
    cm = jnp.max(p.reshape(BT_, CPT_, CHUNK_), axis=2)
    cmax_ref[...] = cm.reshape(1, BT_ // 8, 8 * CPT_)


def _encode(x, W_enc, b_enc, b_dec):
    return pl.pallas_call(
        _enc_body,
        grid=(NHT_, NBT_),
        in_specs=[
            pl.BlockSpec((BT_, D_IN_), lambda h, b: (b, 0)),
            pl.BlockSpec((HT_, D_IN_), lambda h, b: (h, 0)),
            pl.BlockSpec((1, HT_), lambda h, b: (0, h)),
            pl.BlockSpec((1, D_IN_), lambda h, b: (0, 0)),
        ],
        out_specs=[
            pl.BlockSpec((BT_ * CPT_, CHUNK_), lambda h, b: (h * NBT_ + b, 0)),
            pl.BlockSpec((1, BT_ // 8, 8 * CPT_), lambda h, b: (h, b, 0)),
        ],
        out_shape=[
            jax.ShapeDtypeStruct((NHT_ * BATCH_ * CPT_, CHUNK_), jnp.float32),
            jax.ShapeDtypeStruct((NHT_, BATCH_ // 8, 8 * CPT_), jnp.float32),
        ],
    )(x, W_enc, b_enc.reshape(1, HID_), b_dec.reshape(1, D_IN_))


_NEG = -3.0e38
_POS = 3.0e38


def _sc_body(pre2, cmax3h, wdech, bdech, outh,
             cm3_v, cmax_v, idb, chkb, cu, ci, selu, seli, selv,
             wrows, acc, bdv, tmp16, sem1, sem2):
    wid = lax.axis_index("s") * 2 + lax.axis_index("c")
    row0 = wid * RPW_
    rg0 = pl.multiple_of(row0 // 8, RPW_ // 8)
    for h0 in range(NHT_):
        pltpu.sync_copy(cmax3h.at[h0, pl.ds(rg0, RPW_ // 8), :],
                        cm3_v.at[h0])
    pltpu.sync_copy(bdech, bdv)
    iota = lax.iota(jnp.int32, 16)

    def row_body(rl, _carry):
        row = row0 + rl
        vmx = jnp.full((16,), _NEG, dtype=jnp.float32)
        vmn = jnp.full((16,), _POS, dtype=jnp.float32)
        rg = rl // 8
        off = (rl % 8) * 16
        for h in range(NHT_):
            v = cm3_v[h, rg, pl.ds(off, 16)]
            cmax_v[pl.ds(h * 16, 16)] = v
            vmx = jnp.maximum(vmx, v)
            vmn = jnp.minimum(vmn, v)
        for sh in (8, 4, 2, 1):
            tmp16[...] = vmx
            vmx = jnp.maximum(vmx, plsc.load_gather(tmp16, [iota ^ sh]))
            tmp16[...] = vmn
            vmn = jnp.minimum(vmn, plsc.load_gather(tmp16, [iota ^ sh]))
        rmax = vmx[0]
        rmin = vmn[0]

        def th_cnt(t):
            ts = jnp.full((16,), 0.0, dtype=jnp.float32) + t
            a = jnp.zeros((16,), dtype=jnp.int32)
            for h in range(NHT_):
                a = a + plsc.all_reduce_population_count(
                    cmax_v[pl.ds(h * 16, 16)] >= ts)
            return a[0]

        def bisf(_, lohi):
            lo, hi = lohi
            mid = 0.5 * (lo + hi)
            big = th_cnt(mid) >= K_
            return (jnp.where(big, mid, lo), jnp.where(big, hi, mid))

        hi0 = rmax + (jnp.abs(rmax) * 1e-6 + 1e-30)
        th, _unused = lax.fori_loop(0, 20, bisf, (rmin, hi0))
        thv = jnp.full((16,), th, dtype=jnp.float32)

        basev = jnp.full((16,), row * CPT_, dtype=jnp.int32)
        for qv in range(7 * GB_ // 16 + 1):
            idb[pl.ds(qv * 16, 16)] = basev

        def cscan(q, ptr):
            v = cmax_v[pl.ds(q * 16, 16)]
            m = v >= thv
            ids = jnp.full((16,), q * (BATCH_ * CPT_) + row * CPT_,
                           dtype=jnp.int32) + iota
            plsc.store_compressed(idb.at[pl.ds(ptr, 16)], ids, mask=m)
            return ptr + plsc.all_reduce_population_count(m)[0]

        nc = lax.fori_loop(0, NHT_, cscan, 0)
        nb = (nc + GB_ - 1) // GB_

        one = jnp.full((16,), 1, dtype=jnp.int32)
        zero = jnp.zeros((16,), dtype=jnp.int32)

        def select32(cnt):
            cu[pl.ds(cnt, 16)] = jnp.zeros((16,), dtype=jnp.uint32)
            nv = (cnt + 15) // 16

            def cnt_ge(t_u32):
                ts = jnp.full((16,), 0, dtype=jnp.uint32) + t_u32

                def cb(q, a):
                    u = cu[pl.ds(q * 16, 16)]
                    return a + plsc.all_reduce_population_count(u >= ts)

                return lax.fori_loop(0, nv, cb, zero)[0]

            def bis(_, lohi):
                lo, hi = lohi
                mid = lo + (hi - lo) // jnp.uint32(2)
                big = cnt_ge(mid) >= K_
                return (jnp.where(big, mid, lo), jnp.where(big, hi, mid))

            lo, _hi = lax.fori_loop(
                0, 32, bis,
                (jnp.uint32(0), jnp.uint32(0xFFFFFFFF)))
            tsel = jnp.full((16,), 0, dtype=jnp.uint32) + lo
            n_gt = cnt_ge(lo + jnp.uint32(1))

            def ext(q, carry):
                p, neq = carry
                u = cu[pl.ds(q * 16, 16)]
                gi = ci[pl.ds(q * 16, 16)]
                m_gt = u > tsel
                m_eq = u == tsel

                def drop(m):
                    mi = lax.rev(jnp.where(m, one, zero), (0,))
                    last = 15 - plsc.all_reduce_ffs(mi == one)[0]
                    return m & ~(iota == (zero + last))

                m_eq = lax.while_loop(
                    lambda m: plsc.all_reduce_population_count(m)[0] > neq,
                    drop, m_eq)
                m = m_gt | m_eq
                plsc.store_compressed(selu.at[pl.ds(p, 16)], u, mask=m)
                plsc.store_compressed(seli.at[pl.ds(p, 16)], gi, mask=m)
                return (p + plsc.all_reduce_population_count(m)[0],
                        neq - plsc.all_reduce_population_count(m_eq)[0])

            lax.fori_loop(0, nv, ext, (0, K_ - n_gt))

        def batch_body(bb, ptr):
            pltpu.async_copy(pre2.at[idb.at[pl.ds(bb * GB_, GB_)]],
                             chkb, sem1).wait()

            def chunk_body(i, ptr2):
                slot = bb * GB_ + i
                cid = idb[pl.ds(slot, 16)][0]
                over = ptr2 > CAP_

                @pl.when(over)
                def _():
                    select32(ptr2)
                    for t2 in range(K_ // 16):
                        cu[pl.ds(t2 * 16, 16)] = selu[pl.ds(t2 * 16, 16)]
                        ci[pl.ds(t2 * 16, 16)] = seli[pl.ds(t2 * 16, 16)]

                ptr2 = jnp.where(over, K_, ptr2)
                te = thv
                hh = lax.shift_right_logical(cid, 17)
                cc = cid & 15
                jb = (hh * CPT_ + cc) * CHUNK_
                for qq in range(CHUNK_ // 16):
                    v = chkb[i, pl.ds(qq * 16, 16)]
                    m = v >= te
                    s32 = plsc.bitcast(v, jnp.int32)
                    u = plsc.bitcast(
                        jnp.where(s32 < 0, ~s32,
                                  s32 | jnp.int32(-2147483648)),
                        jnp.uint32)
                    gi = jnp.full((16,), jb + qq * 16, dtype=jnp.int32) + iota
                    plsc.store_compressed(cu.at[pl.ds(ptr2, 16)], u, mask=m)
                    plsc.store_compressed(ci.at[pl.ds(ptr2, 16)], gi, mask=m)
                    ptr2 = ptr2 + plsc.all_reduce_population_count(m)[0]
                return ptr2

            return lax.fori_loop(
                0, jnp.minimum(nc - bb * GB_, GB_), chunk_body, ptr)

        cnt = lax.fori_loop(0, nb, batch_body, 0)
        select32(cnt)

        for t2 in range(K_ // 16):
            uu = selu[pl.ds(t2 * 16, 16)]
            s32 = plsc.bitcast(uu, jnp.int32)
            selv[pl.ds(t2 * 16, 16)] = plsc.bitcast(
                jnp.where(s32 < 0, s32 & jnp.int32(0x7FFFFFFF), ~s32),
                jnp.float32)

        pltpu.async_copy(wdech.at[seli.at[pl.ds(0, K_)]], wrows, sem2).wait()
        for j in range(D_IN_ // 16):
            acc[pl.ds(j * 16, 16)] = bdv[pl.ds(j * 16, 16)]

        def dk(k, _):
            vs = plsc.load_gather(selv, [jnp.full((16,), 0, jnp.int32) + k])
            for j in range(D_IN_ // 16):
                w = wrows[k, pl.ds(j * 16, 16)]
                plsc.addupdate(acc.at[pl.ds(j * 16, 16)], vs * w)
            return 0

        lax.fori_loop(0, K_, dk, 0)
        pltpu.sync_copy(acc, outh.at[row])
        return 0

    lax.fori_loop(0, RPW_, row_body, 0)


def _sc_decode(pre2, cmax3, W_dec, b_dec):
    mesh = plsc.VectorSubcoreMesh(core_axis_name="c", subcore_axis_name="s",
                                  num_cores=2, num_subcores=16)
    f = pl.kernel(
        _sc_body,
        out_type=jax.ShapeDtypeStruct((BATCH_, D_IN_), jnp.float32),
        mesh=mesh,
        compiler_params=pltpu.CompilerParams(needs_layout_passes=False),
        scratch_types=[
            pltpu.VMEM((NHT_, RPW_ // 8, 128), jnp.float32),
            pltpu.VMEM((NCHUNK_,), jnp.float32),
            pltpu.VMEM((7 * GB_ + 16,), jnp.int32),
            pltpu.VMEM((GB_, CHUNK_), jnp.float32),
            pltpu.VMEM((CBUF_,), jnp.uint32),
            pltpu.VMEM((CBUF_,), jnp.int32),
            pltpu.VMEM((K_ + 16,), jnp.uint32),
            pltpu.VMEM((K_ + 16,), jnp.int32),
            pltpu.VMEM((K_,), jnp.float32),
            pltpu.VMEM((K_, D_IN_), jnp.float32),
            pltpu.VMEM((D_IN_,), jnp.float32),
            pltpu.VMEM((D_IN_,), jnp.float32),
            pltpu.VMEM((16,), jnp.float32),
            pltpu.SemaphoreType.DMA,
            pltpu.SemaphoreType.DMA,
        ],
    )
    return f(pre2, cmax3, W_dec, b_dec)


@jax.jit
def _run(x, W_enc, b_enc, W_dec, b_dec):
    pre, cmax3 = _encode(x, W_enc, b_enc, b_dec)
    return _sc_decode(pre, cmax3, W_dec, b_dec)


def kernel(x, W_enc, b_enc, W_dec, b_dec):
    return _run(x, W_enc, b_enc, W_dec, b_dec)


# speedup vs baseline: 5.7230x; 1.1166x over previous
import functools

import jax
import jax.numpy as jnp
from jax import lax
from jax.experimental import pallas as pl
from jax.experimental.pallas import tpu as pltpu
from jax.experimental.pallas import tpu_sc as plsc

D_IN_ = 768
HID_ = 24576
BATCH_ = 8192
K_ = 32
CHUNK_ = 128
NCHUNK_ = HID_ // CHUNK_

BT_ = 256
HT_ = 2048
NBT_ = BATCH_ // BT_
NHT_ = HID_ // HT_
CPT_ = HT_ // CHUNK_

NT_ = 64


def _tgrid(i_f32):
    return jnp.where(i_f32 < 16.0, -4.0 + 0.4 * i_f32,
                     2.4 + 0.03 * (i_f32 - 16.0))

NW_ = 32
RPW_ = BATCH_ // NW_
GB_ = 64
CAP_ = 512
CBUF_ = CAP_ + CHUNK_ + 16


def _enc_body(x_ref, we_ref, be_ref, bd_ref, pre_ref, cmax_ref):
    xt = x_ref[...] - bd_ref[...]
    p = lax.dot_general(
        xt, we_ref[...], (((1,), (1,)), ((), ())),
        preferred_element_type=jnp.float32,
        precision=lax.Precision.DEFAULT,
    ) + be_ref[...]
    pre_ref[...] = p.reshape(BT_ * CPT_, CHUNK_)
    cm = jnp.max(p.reshape(BT_, CPT_, CHUNK_), axis=2)
    cmax_ref[...] = cm.reshape(1, BT_ // 8, 8 * CPT_)


def _encode(x, W_enc, b_enc, b_dec):
    return pl.pallas_call(
        _enc_body,
        grid=(NHT_, NBT_),
        in_specs=[
            pl.BlockSpec((BT_, D_IN_), lambda h, b: (b, 0)),
            pl.BlockSpec((HT_, D_IN_), lambda h, b: (h, 0)),
            pl.BlockSpec((1, HT_), lambda h, b: (0, h)),
            pl.BlockSpec((1, D_IN_), lambda h, b: (0, 0)),
        ],
        out_specs=[
            pl.BlockSpec((BT_ * CPT_, CHUNK_), lambda h, b: (h * NBT_ + b, 0)),
            pl.BlockSpec((1, BT_ // 8, 8 * CPT_), lambda h, b: (h, b, 0)),
        ],
        out_shape=[
            jax.ShapeDtypeStruct((NHT_ * BATCH_ * CPT_, CHUNK_), jnp.float32),
            jax.ShapeDtypeStruct((NHT_, BATCH_ // 8, 8 * CPT_), jnp.float32),
        ],
    )(x, W_enc, b_enc.reshape(1, HID_), b_dec.reshape(1, D_IN_))


_NEG = -3.0e38
_POS = 3.0e38


def _sc_body(pre2, cmax3h, wdech, bdech, outh,
             cm3_v, cmax_v, idb, chkb, cu, ci, selu, seli, selv,
             wrows, seli2, acc, bdv, tmp16, sem1, sem2):
    wid = lax.axis_index("s") * 2 + lax.axis_index("c")
    row0 = wid * RPW_
    rg0 = pl.multiple_of(row0 // 8, RPW_ // 8)
    for h0 in range(NHT_):
        pltpu.sync_copy(cmax3h.at[h0, pl.ds(rg0, RPW_ // 8), :],
                        cm3_v.at[h0])
    pltpu.sync_copy(bdech, bdv)
    iota = lax.iota(jnp.int32, 16)

    def row_body(rl, _carry):
        row = row0 + rl
        vmx = jnp.full((16,), _NEG, dtype=jnp.float32)
        vmn = jnp.full((16,), _POS, dtype=jnp.float32)
        rg = rl // 8
        off = (rl % 8) * 16
        for h in range(NHT_):
            v = cm3_v[h, rg, pl.ds(off, 16)]
            cmax_v[pl.ds(h * 16, 16)] = v
            vmx = jnp.maximum(vmx, v)
            vmn = jnp.minimum(vmn, v)
        for sh in (8, 4, 2, 1):
            tmp16[...] = vmx
            vmx = jnp.maximum(vmx, plsc.load_gather(tmp16, [iota ^ sh]))
            tmp16[...] = vmn
            vmn = jnp.minimum(vmn, plsc.load_gather(tmp16, [iota ^ sh]))
        rmax = vmx[0]
        rmin = vmn[0]

        def th_cnt(t):
            ts = jnp.full((16,), 0.0, dtype=jnp.float32) + t
            a = jnp.zeros((16,), dtype=jnp.int32)
            for h in range(NHT_):
                a = a + plsc.all_reduce_population_count(
                    cmax_v[pl.ds(h * 16, 16)] >= ts)
            return a[0]

        def bisf(_, lohi):
            lo, hi = lohi
            mid = 0.5 * (lo + hi)
            big = th_cnt(mid) >= K_
            return (jnp.where(big, mid, lo), jnp.where(big, hi, mid))

        hi0 = rmax + (jnp.abs(rmax) * 1e-6 + 1e-30)
        th, _unused = lax.fori_loop(0, 14, bisf, (rmin, hi0))
        thv = jnp.full((16,), th, dtype=jnp.float32)

        basev = jnp.full((16,), row * CPT_, dtype=jnp.int32)
        for qv in range(7 * GB_ // 16 + 1):
            idb[pl.ds(qv * 16, 16)] = basev

        def cscan(q, ptr):
            v = cmax_v[pl.ds(q * 16, 16)]
            m = v >= thv
            ids = jnp.full((16,), q * (BATCH_ * CPT_) + row * CPT_,
                           dtype=jnp.int32) + iota
            plsc.store_compressed(idb.at[pl.ds(ptr, 16)], ids, mask=m)
            return ptr + plsc.all_reduce_population_count(m)[0]

        nc = lax.fori_loop(0, NHT_, cscan, 0)
        nb = (nc + GB_ - 1) // GB_

        one = jnp.full((16,), 1, dtype=jnp.int32)
        zero = jnp.zeros((16,), dtype=jnp.int32)

        def select32(cnt):
            cu[pl.ds(cnt, 16)] = jnp.zeros((16,), dtype=jnp.uint32)
            nv = (cnt + 15) // 16

            def cnt_ge(t_u32):
                ts = jnp.full((16,), 0, dtype=jnp.uint32) + t_u32

                def cb(q, a):
                    u = cu[pl.ds(q * 16, 16)]
                    return a + plsc.all_reduce_population_count(u >= ts)

                return lax.fori_loop(0, nv, cb, zero)[0]

            s_th = lax.bitcast_convert_type(th, jnp.int32)
            lo0 = lax.bitcast_convert_type(
                jnp.where(s_th < 0, ~s_th, s_th | jnp.int32(-2147483648)),
                jnp.uint32)
            s_mx = lax.bitcast_convert_type(rmax, jnp.int32)
            hi0b = lax.bitcast_convert_type(
                jnp.where(s_mx < 0, ~s_mx, s_mx | jnp.int32(-2147483648)),
                jnp.uint32) + jnp.uint32(1)

            def bis(lohi):
                lo, hi = lohi
                mid = lo + (hi - lo) // jnp.uint32(2)
                big = cnt_ge(mid) >= K_
                return (jnp.where(big, mid, lo), jnp.where(big, hi, mid))

            lo, _hi = lax.while_loop(
                lambda lohi: lohi[1] - lohi[0] > jnp.uint32(1),
                bis, (lo0, hi0b))
            tsel = jnp.full((16,), 0, dtype=jnp.uint32) + lo
            n_gt = cnt_ge(lo + jnp.uint32(1))

            def ext(q, carry):
                p, neq = carry
                u = cu[pl.ds(q * 16, 16)]
                gi = ci[pl.ds(q * 16, 16)]
                m_gt = u > tsel
                m_eq = u == tsel

                def drop(m):
                    mi = lax.rev(jnp.where(m, one, zero), (0,))
                    last = 15 - plsc.all_reduce_ffs(mi == one)[0]
                    return m & ~(iota == (zero + last))

                m_eq = lax.while_loop(
                    lambda m: plsc.all_reduce_population_count(m)[0] > neq,
                    drop, m_eq)
                m = m_gt | m_eq
                plsc.store_compressed(selu.at[pl.ds(p, 16)], u, mask=m)
                plsc.store_compressed(seli.at[pl.ds(p, 16)], gi, mask=m)
                return (p + plsc.all_reduce_population_count(m)[0],
                        neq - plsc.all_reduce_population_count(m_eq)[0])

            lax.fori_loop(0, nv, ext, (0, K_ - n_gt))

        def batch_body(bb, ptr):
            pltpu.async_copy(pre2.at[idb.at[pl.ds(bb * GB_, GB_)]],
                             chkb, sem1).wait()

            def chunk_body(i, ptr2):
                slot = bb * GB_ + i
                cid = idb[pl.ds(slot, 16)][0]
                over = ptr2 > CAP_

                @pl.when(over)
                def _():
                    select32(ptr2)
                    for t2 in range(K_ // 16):
                        cu[pl.ds(t2 * 16, 16)] = selu[pl.ds(t2 * 16, 16)]
                        ci[pl.ds(t2 * 16, 16)] = seli[pl.ds(t2 * 16, 16)]

                ptr2 = jnp.where(over, K_, ptr2)
                te = thv
                hh = lax.shift_right_logical(cid, 17)
                cc = cid & 15
                jb = (hh * CPT_ + cc) * CHUNK_
                for qq in range(CHUNK_ // 16):
                    v = chkb[i, pl.ds(qq * 16, 16)]
                    m = v >= te
                    s32 = plsc.bitcast(v, jnp.int32)
                    u = plsc.bitcast(
                        jnp.where(s32 < 0, ~s32,
                                  s32 | jnp.int32(-2147483648)),
                        jnp.uint32)
                    gi = jnp.full((16,), jb + qq * 16, dtype=jnp.int32) + iota
                    plsc.store_compressed(cu.at[pl.ds(ptr2, 16)], u, mask=m)
                    plsc.store_compressed(ci.at[pl.ds(ptr2, 16)], gi, mask=m)
                    ptr2 = ptr2 + plsc.all_reduce_population_count(m)[0]
                return ptr2

            return lax.fori_loop(
                0, jnp.minimum(nc - bb * GB_, GB_), chunk_body, ptr)

        cnt = lax.fori_loop(0, nb, batch_body, 0)
        select32(cnt)

        par = rl & 1
        for t2 in range(K_ // 16):
            uu = selu[pl.ds(t2 * 16, 16)]
            s32 = plsc.bitcast(uu, jnp.int32)
            selv[pl.ds(par * 64 + t2 * 16, 16)] = plsc.bitcast(
                jnp.where(s32 < 0, s32 & jnp.int32(0x7FFFFFFF), ~s32),
                jnp.float32)
            seli2[pl.ds(par * 64 + t2 * 16, 16)] = seli[pl.ds(t2 * 16, 16)]
        pltpu.async_copy(wdech.at[seli2.at[pl.ds(par * 64, K_)]],
                         wrows.at[pl.ds(par * K_, K_)], sem2)

        @pl.when(rl > 0)
        def _():
            _decode_row(1 - par, row - 1)

        return 0

    def _decode_row(p, orow):
        pltpu.make_async_copy(
            wdech.at[seli2.at[pl.ds(p * 64, K_)]],
            wrows.at[pl.ds(p * K_, K_)], sem2).wait()
        for j in range(D_IN_ // 16):
            acc[pl.ds(j * 16, 16)] = bdv[pl.ds(j * 16, 16)]

        def dk(k, _):
            vs = plsc.load_gather(
                selv, [jnp.full((16,), 0, jnp.int32) + (p * 64 + k)])
            for j in range(D_IN_ // 16):
                w = wrows[p * K_ + k, pl.ds(j * 16, 16)]
                plsc.addupdate(acc.at[pl.ds(j * 16, 16)], vs * w)
            return 0

        lax.fori_loop(0, K_, dk, 0)
        pltpu.sync_copy(acc, outh.at[orow])

    lax.fori_loop(0, RPW_, row_body, 0)
    _decode_row((RPW_ - 1) & 1, row0 + RPW_ - 1)


def _sc_decode(pre2, cmax3, W_dec, b_dec):
    mesh = plsc.VectorSubcoreMesh(core_axis_name="c", subcore_axis_name="s",
                                  num_cores=2, num_subcores=16)
    f = pl.kernel(
        _sc_body,
        out_type=jax.ShapeDtypeStruct((BATCH_, D_IN_), jnp.float32),
        mesh=mesh,
        compiler_params=pltpu.CompilerParams(needs_layout_passes=False),
        scratch_types=[
            pltpu.VMEM((NHT_, RPW_ // 8, 128), jnp.float32),
            pltpu.VMEM((NCHUNK_,), jnp.float32),
            pltpu.VMEM((7 * GB_ + 16,), jnp.int32),
            pltpu.VMEM((GB_, CHUNK_), jnp.float32),
            pltpu.VMEM((CBUF_,), jnp.uint32),
            pltpu.VMEM((CBUF_,), jnp.int32),
            pltpu.VMEM((K_ + 16,), jnp.uint32),
            pltpu.VMEM((K_ + 16,), jnp.int32),
            pltpu.VMEM((128,), jnp.float32),
            pltpu.VMEM((2 * K_, D_IN_), jnp.float32),
            pltpu.VMEM((128,), jnp.int32),
            pltpu.VMEM((D_IN_,), jnp.float32),
            pltpu.VMEM((D_IN_,), jnp.float32),
            pltpu.VMEM((16,), jnp.float32),
            pltpu.SemaphoreType.DMA,
            pltpu.SemaphoreType.DMA,
        ],
    )
    return f(pre2, cmax3, W_dec, b_dec)


@jax.jit
def _run(x, W_enc, b_enc, W_dec, b_dec):
    pre, cmax3 = _encode(x, W_enc, b_enc, b_dec)
    return _sc_decode(pre, cmax3, W_dec, b_dec)


def kernel(x, W_enc, b_enc, W_dec, b_dec):
    return _run(x, W_enc, b_enc, W_dec, b_dec)


# speedup vs baseline: 5.7394x; 1.0029x over previous
import functools

import jax
import jax.numpy as jnp
from jax import lax
from jax.experimental import pallas as pl
from jax.experimental.pallas import tpu as pltpu
from jax.experimental.pallas import tpu_sc as plsc

D_IN_ = 768
HID_ = 24576
BATCH_ = 8192
K_ = 32
CHUNK_ = 128
NCHUNK_ = HID_ // CHUNK_

BT_ = 256
HT_ = 2048
NBT_ = BATCH_ // BT_
NHT_ = HID_ // HT_
CPT_ = HT_ // CHUNK_

NT_ = 64


def _tgrid(i_f32):
    return jnp.where(i_f32 < 16.0, -4.0 + 0.4 * i_f32,
                     2.4 + 0.03 * (i_f32 - 16.0))

NW_ = 32
RPW_ = BATCH_ // NW_
GB_ = 64
CAP_ = 512
CBUF_ = CAP_ + CHUNK_ + 16


def _enc_body(x_ref, we_ref, be_ref, bd_ref, pre_ref, cmax_ref):
    xt = x_ref[...] - bd_ref[...]
    p = lax.dot_general(
        xt, we_ref[...], (((1,), (1,)), ((), ())),
        preferred_element_type=jnp.float32,
        precision=lax.Precision.DEFAULT,
    ) + be_ref[...]
    pre_ref[...] = p.reshape(BT_ * CPT_, CHUNK_)
    cm = jnp.max(p.reshape(BT_, CPT_, CHUNK_), axis=2)
    cmax_ref[...] = cm.reshape(1, BT_ // 8, 8 * CPT_)


def _encode(x, W_enc, b_enc, b_dec):
    return pl.pallas_call(
        _enc_body,
        grid=(NHT_, NBT_),
        in_specs=[
            pl.BlockSpec((BT_, D_IN_), lambda h, b: (b, 0)),
            pl.BlockSpec((HT_, D_IN_), lambda h, b: (h, 0)),
            pl.BlockSpec((1, HT_), lambda h, b: (0, h)),
            pl.BlockSpec((1, D_IN_), lambda h, b: (0, 0)),
        ],
        out_specs=[
            pl.BlockSpec((BT_ * CPT_, CHUNK_), lambda h, b: (h * NBT_ + b, 0)),
            pl.BlockSpec((1, BT_ // 8, 8 * CPT_), lambda h, b: (h, b, 0)),
        ],
        out_shape=[
            jax.ShapeDtypeStruct((NHT_ * BATCH_ * CPT_, CHUNK_), jnp.float32),
            jax.ShapeDtypeStruct((NHT_, BATCH_ // 8, 8 * CPT_), jnp.float32),
        ],
    )(x, W_enc, b_enc.reshape(1, HID_), b_dec.reshape(1, D_IN_))


_NEG = -3.0e38
_POS = 3.0e38


def _sc_body(pre2, cmax3h, wdech, bdech, outh,
             cm3_v, idb, chkb, cu, ci, selu, seli, selv,
             wrows, seli2, acc, bdv, tmp16, sem1, sem2):
    wid = lax.axis_index("s") * 2 + lax.axis_index("c")
    row0 = wid * RPW_
    rg0 = pl.multiple_of(row0 // 8, RPW_ // 8)
    for h0 in range(NHT_):
        pltpu.sync_copy(cmax3h.at[h0, pl.ds(rg0, RPW_ // 8), :],
                        cm3_v.at[h0])
    pltpu.sync_copy(bdech, bdv)
    iota = lax.iota(jnp.int32, 16)

    def row_body(rl, _carry):
        row = row0 + rl
        vmx = jnp.full((16,), _NEG, dtype=jnp.float32)
        vmn = jnp.full((16,), _POS, dtype=jnp.float32)
        rg = rl // 8
        off = (rl % 8) * 16
        cmvecs = []
        for h in range(NHT_):
            v = cm3_v[h, rg, pl.ds(off, 16)]
            cmvecs.append(v)
            vmx = jnp.maximum(vmx, v)
            vmn = jnp.minimum(vmn, v)
        for sh in (8, 4, 2, 1):
            tmp16[...] = vmx
            vmx = jnp.maximum(vmx, plsc.load_gather(tmp16, [iota ^ sh]))
            tmp16[...] = vmn
            vmn = jnp.minimum(vmn, plsc.load_gather(tmp16, [iota ^ sh]))
        rmax = vmx[0]
        rmin = vmn[0]

        def th_cnt(t):
            ts = jnp.full((16,), 0.0, dtype=jnp.float32) + t
            a = jnp.zeros((16,), dtype=jnp.int32)
            for h in range(NHT_):
                a = a + plsc.all_reduce_population_count(cmvecs[h] >= ts)
            return a[0]

        def bisf(_, lohi):
            lo, hi = lohi
            mid = 0.5 * (lo + hi)
            big = th_cnt(mid) >= K_
            return (jnp.where(big, mid, lo), jnp.where(big, hi, mid))

        hi0 = rmax + (jnp.abs(rmax) * 1e-6 + 1e-30)
        th, _unused = lax.fori_loop(0, 14, bisf, (rmin, hi0))
        thv = jnp.full((16,), th, dtype=jnp.float32)

        basev = jnp.full((16,), row * CPT_, dtype=jnp.int32)
        for qv in range(7 * GB_ // 16 + 1):
            idb[pl.ds(qv * 16, 16)] = basev

        nc = 0
        for q in range(NHT_):
            m = cmvecs[q] >= thv
            ids = jnp.full((16,), q * (BATCH_ * CPT_) + row * CPT_,
                           dtype=jnp.int32) + iota
            plsc.store_compressed(idb.at[pl.ds(nc, 16)], ids, mask=m)
            nc = nc + plsc.all_reduce_population_count(m)[0]

        nb = (nc + GB_ - 1) // GB_

        one = jnp.full((16,), 1, dtype=jnp.int32)
        zero = jnp.zeros((16,), dtype=jnp.int32)

        def select32(cnt):
            nv = (cnt + 15) // 16

            def monoq(q, _):
                s32 = plsc.bitcast(cu[pl.ds(q * 16, 16)], jnp.int32)
                cu[pl.ds(q * 16, 16)] = plsc.bitcast(
                    jnp.where(s32 < 0, ~s32, s32 | jnp.int32(-2147483648)),
                    jnp.uint32)
                return 0

            lax.fori_loop(0, nv, monoq, 0)
            cu[pl.ds(cnt, 16)] = jnp.zeros((16,), dtype=jnp.uint32)

            def cnt_ge(t_u32):
                ts = jnp.full((16,), 0, dtype=jnp.uint32) + t_u32

                def cb(q, a):
                    u = cu[pl.ds(q * 16, 16)]
                    return a + plsc.all_reduce_population_count(u >= ts)

                return lax.fori_loop(0, nv, cb, zero)[0]

            s_th = lax.bitcast_convert_type(th, jnp.int32)
            lo0 = lax.bitcast_convert_type(
                jnp.where(s_th < 0, ~s_th, s_th | jnp.int32(-2147483648)),
                jnp.uint32)
            s_mx = lax.bitcast_convert_type(rmax, jnp.int32)
            hi0b = lax.bitcast_convert_type(
                jnp.where(s_mx < 0, ~s_mx, s_mx | jnp.int32(-2147483648)),
                jnp.uint32) + jnp.uint32(1)

            def bis(lohi):
                lo, hi = lohi
                mid = lo + (hi - lo) // jnp.uint32(2)
                big = cnt_ge(mid) >= K_
                return (jnp.where(big, mid, lo), jnp.where(big, hi, mid))

            lo, _hi = lax.while_loop(
                lambda lohi: lohi[1] - lohi[0] > jnp.uint32(1),
                bis, (lo0, hi0b))
            tsel = jnp.full((16,), 0, dtype=jnp.uint32) + lo
            n_gt = cnt_ge(lo + jnp.uint32(1))

            def ext(q, carry):
                p, neq = carry
                u = cu[pl.ds(q * 16, 16)]
                gi = ci[pl.ds(q * 16, 16)]
                m_gt = u > tsel
                m_eq = u == tsel

                def drop(m):
                    mi = lax.rev(jnp.where(m, one, zero), (0,))
                    last = 15 - plsc.all_reduce_ffs(mi == one)[0]
                    return m & ~(iota == (zero + last))

                m_eq = lax.while_loop(
                    lambda m: plsc.all_reduce_population_count(m)[0] > neq,
                    drop, m_eq)
                m = m_gt | m_eq
                plsc.store_compressed(selu.at[pl.ds(p, 16)], u, mask=m)
                plsc.store_compressed(seli.at[pl.ds(p, 16)], gi, mask=m)
                return (p + plsc.all_reduce_population_count(m)[0],
                        neq - plsc.all_reduce_population_count(m_eq)[0])

            lax.fori_loop(0, nv, ext, (0, K_ - n_gt))

        def batch_body(bb, ptr):
            pltpu.async_copy(pre2.at[idb.at[pl.ds(bb * GB_, GB_)]],
                             chkb, sem1).wait()

            def chunk_body(i, ptr2):
                slot = bb * GB_ + i
                cid = idb[pl.ds(slot, 16)][0]
                over = ptr2 > CAP_

                @pl.when(over)
                def _():
                    select32(ptr2)
                    for t2 in range(K_ // 16):
                        us = plsc.bitcast(selu[pl.ds(t2 * 16, 16)], jnp.int32)
                        cu[pl.ds(t2 * 16, 16)] = plsc.bitcast(
                            jnp.where(us < 0, us & jnp.int32(0x7FFFFFFF),
                                      ~us), jnp.uint32)
                        ci[pl.ds(t2 * 16, 16)] = seli[pl.ds(t2 * 16, 16)]

                ptr2 = jnp.where(over, K_, ptr2)
                te = thv
                hh = lax.shift_right_logical(cid, 17)
                cc = cid & 15
                jb = (hh * CPT_ + cc) * CHUNK_
                for qq in range(CHUNK_ // 16):
                    v = chkb[i, pl.ds(qq * 16, 16)]
                    m = v >= te
                    u = plsc.bitcast(v, jnp.uint32)
                    gi = jnp.full((16,), jb + qq * 16, dtype=jnp.int32) + iota
                    plsc.store_compressed(cu.at[pl.ds(ptr2, 16)], u, mask=m)
                    plsc.store_compressed(ci.at[pl.ds(ptr2, 16)], gi, mask=m)
                    ptr2 = ptr2 + plsc.all_reduce_population_count(m)[0]
                return ptr2

            return lax.fori_loop(
                0, jnp.minimum(nc - bb * GB_, GB_), chunk_body, ptr)

        cnt = lax.fori_loop(0, nb, batch_body, 0)
        select32(cnt)

        par = rl & 1
        for t2 in range(K_ // 16):
            uu = selu[pl.ds(t2 * 16, 16)]
            s32 = plsc.bitcast(uu, jnp.int32)
            selv[pl.ds(par * 64 + t2 * 16, 16)] = plsc.bitcast(
                jnp.where(s32 < 0, s32 & jnp.int32(0x7FFFFFFF), ~s32),
                jnp.float32)
            seli2[pl.ds(par * 64 + t2 * 16, 16)] = seli[pl.ds(t2 * 16, 16)]
        pltpu.async_copy(wdech.at[seli2.at[pl.ds(par * 64, K_)]],
                         wrows.at[pl.ds(par * K_, K_)], sem2)

        @pl.when(rl > 0)
        def _():
            _decode_row(1 - par, row - 1)

        return 0

    def _decode_row(p, orow):
        pltpu.make_async_copy(
            wdech.at[seli2.at[pl.ds(p * 64, K_)]],
            wrows.at[pl.ds(p * K_, K_)], sem2).wait()
        for j in range(D_IN_ // 16):
            acc[pl.ds(j * 16, 16)] = bdv[pl.ds(j * 16, 16)]

        def dk(k, _):
            vs = plsc.load_gather(
                selv, [jnp.full((16,), 0, jnp.int32) + (p * 64 + k)])
            for j in range(D_IN_ // 16):
                w = wrows[p * K_ + k, pl.ds(j * 16, 16)]
                plsc.addupdate(acc.at[pl.ds(j * 16, 16)], vs * w)
            return 0

        lax.fori_loop(0, K_, dk, 0)
        pltpu.sync_copy(acc, outh.at[orow])

    lax.fori_loop(0, RPW_, row_body, 0)
    _decode_row((RPW_ - 1) & 1, row0 + RPW_ - 1)


def _sc_decode(pre2, cmax3, W_dec, b_dec):
    mesh = plsc.VectorSubcoreMesh(core_axis_name="c", subcore_axis_name="s",
                                  num_cores=2, num_subcores=16)
    f = pl.kernel(
        _sc_body,
        out_type=jax.ShapeDtypeStruct((BATCH_, D_IN_), jnp.float32),
        mesh=mesh,
        compiler_params=pltpu.CompilerParams(needs_layout_passes=False),
        scratch_types=[
            pltpu.VMEM((NHT_, RPW_ // 8, 128), jnp.float32),
            pltpu.VMEM((7 * GB_ + 16,), jnp.int32),
            pltpu.VMEM((GB_, CHUNK_), jnp.float32),
            pltpu.VMEM((CBUF_,), jnp.uint32),
            pltpu.VMEM((CBUF_,), jnp.int32),
            pltpu.VMEM((K_ + 16,), jnp.uint32),
            pltpu.VMEM((K_ + 16,), jnp.int32),
            pltpu.VMEM((128,), jnp.float32),
            pltpu.VMEM((2 * K_, D_IN_), jnp.float32),
            pltpu.VMEM((128,), jnp.int32),
            pltpu.VMEM((D_IN_,), jnp.float32),
            pltpu.VMEM((D_IN_,), jnp.float32),
            pltpu.VMEM((16,), jnp.float32),
            pltpu.SemaphoreType.DMA,
            pltpu.SemaphoreType.DMA,
        ],
    )
    return f(pre2, cmax3, W_dec, b_dec)


@jax.jit
def _run(x, W_enc, b_enc, W_dec, b_dec):
    pre, cmax3 = _encode(x, W_enc, b_enc, b_dec)
    return _sc_decode(pre, cmax3, W_dec, b_dec)


def kernel(x, W_enc, b_enc, W_dec, b_dec):
    return _run(x, W_enc, b_enc, W_dec, b_dec)


# speedup vs baseline: 8.6453x; 1.5063x over previous
import functools

import jax
import jax.numpy as jnp
from jax import lax
from jax.experimental import pallas as pl
from jax.experimental.pallas import tpu as pltpu
from jax.experimental.pallas import tpu_sc as plsc

D_IN_ = 768
HID_ = 24576
BATCH_ = 8192
K_ = 32
CHUNK_ = 128
NCHUNK_ = HID_ // CHUNK_

BT_ = 256
HT_ = 2048
NBT_ = BATCH_ // BT_
NHT_ = HID_ // HT_
CPT_ = HT_ // CHUNK_

NT_ = 64


def _tgrid(i_f32):
    return jnp.where(i_f32 < 16.0, -4.0 + 0.4 * i_f32,
                     2.4 + 0.03 * (i_f32 - 16.0))

NW_ = 32
RPW_ = BATCH_ // NW_
GB_ = 64
CAP_ = 512
CBUF_ = CAP_ + CHUNK_ + 16


def _enc_body(x_ref, we_ref, be_ref, bd_ref, pre_ref, cmax_ref):
    xt = x_ref[...] - bd_ref[...]
    p = lax.dot_general(
        xt, we_ref[...], (((1,), (1,)), ((), ())),
        preferred_element_type=jnp.float32,
        precision=lax.Precision.DEFAULT,
    ) + be_ref[...]
    pre_ref[...] = p.reshape(BT_ * CPT_, CHUNK_)
    cm = jnp.max(p.reshape(BT_, CPT_, CHUNK_), axis=2)
    cmax_ref[...] = cm.reshape(1, BT_ // 8, 8 * CPT_)


def _encode(x, W_enc, b_enc, b_dec):
    return pl.pallas_call(
        _enc_body,
        grid=(NHT_, NBT_),
        in_specs=[
            pl.BlockSpec((BT_, D_IN_), lambda h, b: (b, 0)),
            pl.BlockSpec((HT_, D_IN_), lambda h, b: (h, 0)),
            pl.BlockSpec((1, HT_), lambda h, b: (0, h)),
            pl.BlockSpec((1, D_IN_), lambda h, b: (0, 0)),
        ],
        out_specs=[
            pl.BlockSpec((BT_ * CPT_, CHUNK_), lambda h, b: (h * NBT_ + b, 0)),
            pl.BlockSpec((1, BT_ // 8, 8 * CPT_), lambda h, b: (h, b, 0)),
        ],
        out_shape=[
            jax.ShapeDtypeStruct((NHT_ * BATCH_ * CPT_, CHUNK_), jnp.float32),
            jax.ShapeDtypeStruct((NHT_, BATCH_ // 8, 8 * CPT_), jnp.float32),
        ],
    )(x, W_enc, b_enc.reshape(1, HID_), b_dec.reshape(1, D_IN_))


_NEG = -3.0e38
_POS = 3.0e38


def _sc_body(pre2, cmax3h, wdech, bdech, outh,
             cm3_v, idb, chkb, cu, ci, selu, seli, selv,
             wrows, seli2, acc, bdv, tmp16, sem1, sem2):
    wid = lax.axis_index("s") * 2 + lax.axis_index("c")
    row0 = wid * RPW_
    rg0 = pl.multiple_of(row0 // 8, RPW_ // 8)
    for h0 in range(NHT_):
        pltpu.sync_copy(cmax3h.at[h0, pl.ds(rg0, RPW_ // 8), :],
                        cm3_v.at[h0])
    pltpu.sync_copy(bdech, bdv)
    iota = lax.iota(jnp.int32, 16)

    def row_body(rl, _carry):
        row = row0 + rl
        vmx = jnp.full((16,), _NEG, dtype=jnp.float32)
        vmn = jnp.full((16,), _POS, dtype=jnp.float32)
        rg = rl // 8
        off = (rl % 8) * 16
        cmvecs = []
        for h in range(NHT_):
            v = cm3_v[h, rg, pl.ds(off, 16)]
            cmvecs.append(v)
            vmx = jnp.maximum(vmx, v)
            vmn = jnp.minimum(vmn, v)
        for sh in (8, 4, 2, 1):
            tmp16[...] = vmx
            vmx = jnp.maximum(vmx, plsc.load_gather(tmp16, [iota ^ sh]))
            tmp16[...] = vmn
            vmn = jnp.minimum(vmn, plsc.load_gather(tmp16, [iota ^ sh]))
        rmax = vmx[0]
        rmin = vmn[0]

        def th_cnt(t):
            ts = jnp.full((16,), 0.0, dtype=jnp.float32) + t
            a = jnp.zeros((16,), dtype=jnp.int32)
            for h in range(NHT_):
                a = a + plsc.all_reduce_population_count(cmvecs[h] >= ts)
            return a[0]

        def bisf(_, lohi):
            lo, hi = lohi
            mid = 0.5 * (lo + hi)
            big = th_cnt(mid) >= K_
            return (jnp.where(big, mid, lo), jnp.where(big, hi, mid))

        hi0 = rmax + (jnp.abs(rmax) * 1e-6 + 1e-30)
        th, _unused = lax.fori_loop(0, 14, bisf, (rmin, hi0))
        thv = jnp.full((16,), th, dtype=jnp.float32)

        basev = jnp.full((16,), row * CPT_, dtype=jnp.int32)
        for qv in range(7 * GB_ // 16 + 1):
            idb[pl.ds(qv * 16, 16)] = basev

        nc = 0
        for q in range(NHT_):
            m = cmvecs[q] >= thv
            ids = jnp.full((16,), q * (BATCH_ * CPT_) + row * CPT_,
                           dtype=jnp.int32) + iota
            plsc.store_compressed(idb.at[pl.ds(nc, 16)], ids, mask=m)
            nc = nc + plsc.all_reduce_population_count(m)[0]

        nb = (nc + GB_ - 1) // GB_

        one = jnp.full((16,), 1, dtype=jnp.int32)
        zero = jnp.zeros((16,), dtype=jnp.int32)

        def select32(cnt):
            nv = (cnt + 15) // 16

            def monoq(q, _):
                s32 = plsc.bitcast(cu[pl.ds(q * 16, 16)], jnp.int32)
                cu[pl.ds(q * 16, 16)] = plsc.bitcast(
                    jnp.where(s32 < 0, ~s32, s32 | jnp.int32(-2147483648)),
                    jnp.uint32)
                return 0

            lax.fori_loop(0, nv, monoq, 0)
            cu[pl.ds(cnt, 16)] = jnp.zeros((16,), dtype=jnp.uint32)

            def cnt_ge(t_u32):
                ts = jnp.full((16,), 0, dtype=jnp.uint32) + t_u32

                def cb(q, a):
                    u = cu[pl.ds(q * 16, 16)]
                    return a + plsc.all_reduce_population_count(u >= ts)

                return lax.fori_loop(0, nv, cb, zero)[0]

            s_th = lax.bitcast_convert_type(th, jnp.int32)
            lo0 = lax.bitcast_convert_type(
                jnp.where(s_th < 0, ~s_th, s_th | jnp.int32(-2147483648)),
                jnp.uint32)
            s_mx = lax.bitcast_convert_type(rmax, jnp.int32)
            hi0b = lax.bitcast_convert_type(
                jnp.where(s_mx < 0, ~s_mx, s_mx | jnp.int32(-2147483648)),
                jnp.uint32) + jnp.uint32(1)

            def bis(lohi):
                lo, hi = lohi
                mid = lo + (hi - lo) // jnp.uint32(2)
                big = cnt_ge(mid) >= K_
                return (jnp.where(big, mid, lo), jnp.where(big, hi, mid))

            lo, _hi = lax.while_loop(
                lambda lohi: lohi[1] - lohi[0] > jnp.uint32(1),
                bis, (lo0, hi0b))
            tsel = jnp.full((16,), 0, dtype=jnp.uint32) + lo
            n_gt = cnt_ge(lo + jnp.uint32(1))

            def ext(q, carry):
                p, neq = carry
                u = cu[pl.ds(q * 16, 16)]
                gi = ci[pl.ds(q * 16, 16)]
                m_gt = u > tsel
                m_eq = u == tsel

                def drop(m):
                    mi = lax.rev(jnp.where(m, one, zero), (0,))
                    last = 15 - plsc.all_reduce_ffs(mi == one)[0]
                    return m & ~(iota == (zero + last))

                m_eq = lax.while_loop(
                    lambda m: plsc.all_reduce_population_count(m)[0] > neq,
                    drop, m_eq)
                m = m_gt | m_eq
                plsc.store_compressed(selu.at[pl.ds(p, 16)], u, mask=m)
                plsc.store_compressed(seli.at[pl.ds(p, 16)], gi, mask=m)
                return (p + plsc.all_reduce_population_count(m)[0],
                        neq - plsc.all_reduce_population_count(m_eq)[0])

            lax.fori_loop(0, nv, ext, (0, K_ - n_gt))

        def batch_body(bb, ptr):
            pltpu.async_copy(pre2.at[idb.at[pl.ds(bb * GB_, GB_)]],
                             chkb, sem1).wait()

            def chunk_body(i, ptr2):
                slot = bb * GB_ + i
                cid = idb[pl.ds(slot, 16)][0]
                over = ptr2 > CAP_

                @pl.when(over)
                def _():
                    select32(ptr2)
                    for t2 in range(K_ // 16):
                        us = plsc.bitcast(selu[pl.ds(t2 * 16, 16)], jnp.int32)
                        cu[pl.ds(t2 * 16, 16)] = plsc.bitcast(
                            jnp.where(us < 0, us & jnp.int32(0x7FFFFFFF),
                                      ~us), jnp.uint32)
                        ci[pl.ds(t2 * 16, 16)] = seli[pl.ds(t2 * 16, 16)]

                ptr2 = jnp.where(over, K_, ptr2)
                te = thv
                hh = lax.shift_right_logical(cid, 17)
                cc = cid & 15
                jb = (hh * CPT_ + cc) * CHUNK_
                for qq in range(CHUNK_ // 16):
                    v = chkb[i, pl.ds(qq * 16, 16)]
                    m = v >= te
                    u = plsc.bitcast(v, jnp.uint32)
                    gi = jnp.full((16,), jb + qq * 16, dtype=jnp.int32) + iota
                    plsc.store_compressed(cu.at[pl.ds(ptr2, 16)], u, mask=m)
                    plsc.store_compressed(ci.at[pl.ds(ptr2, 16)], gi, mask=m)
                    ptr2 = ptr2 + plsc.all_reduce_population_count(m)[0]
                return ptr2

            return lax.fori_loop(
                0, jnp.minimum(nc - bb * GB_, GB_), chunk_body, ptr)

        cnt = lax.fori_loop(0, nb, batch_body, 0)
        select32(cnt)

        par = rl & 1
        for t2 in range(K_ // 16):
            uu = selu[pl.ds(t2 * 16, 16)]
            s32 = plsc.bitcast(uu, jnp.int32)
            selv[pl.ds(par * 64 + t2 * 16, 16)] = plsc.bitcast(
                jnp.where(s32 < 0, s32 & jnp.int32(0x7FFFFFFF), ~s32),
                jnp.float32)
            seli2[pl.ds(par * 64 + t2 * 16, 16)] = seli[pl.ds(t2 * 16, 16)]
        pltpu.async_copy(wdech.at[seli2.at[pl.ds(par * 64, K_)]],
                         wrows.at[pl.ds(par * K_, K_)], sem2)

        @pl.when(rl > 0)
        def _():
            _decode_row(1 - par, row - 1)

        return 0

    def _decode_row(p, orow):
        pltpu.make_async_copy(
            wdech.at[seli2.at[pl.ds(p * 64, K_)]],
            wrows.at[pl.ds(p * K_, K_)], sem2).wait()

        def dk(k, accs):
            vs = plsc.load_gather(
                selv, [jnp.full((16,), 0, jnp.int32) + (p * 64 + k)])
            return tuple(
                a + vs * wrows[p * K_ + k, pl.ds(j * 16, 16)]
                for j, a in enumerate(accs))

        accs0 = tuple(bdv[pl.ds(j * 16, 16)] for j in range(D_IN_ // 16))
        accs = lax.fori_loop(0, K_, dk, accs0)
        for j in range(D_IN_ // 16):
            acc[pl.ds(j * 16, 16)] = accs[j]
        pltpu.sync_copy(acc, outh.at[orow])

    lax.fori_loop(0, RPW_, row_body, 0)
    _decode_row((RPW_ - 1) & 1, row0 + RPW_ - 1)


def _sc_decode(pre2, cmax3, W_dec, b_dec):
    mesh = plsc.VectorSubcoreMesh(core_axis_name="c", subcore_axis_name="s",
                                  num_cores=2, num_subcores=16)
    f = pl.kernel(
        _sc_body,
        out_type=jax.ShapeDtypeStruct((BATCH_, D_IN_), jnp.float32),
        mesh=mesh,
        compiler_params=pltpu.CompilerParams(needs_layout_passes=False),
        scratch_types=[
            pltpu.VMEM((NHT_, RPW_ // 8, 128), jnp.float32),
            pltpu.VMEM((7 * GB_ + 16,), jnp.int32),
            pltpu.VMEM((GB_, CHUNK_), jnp.float32),
            pltpu.VMEM((CBUF_,), jnp.uint32),
            pltpu.VMEM((CBUF_,), jnp.int32),
            pltpu.VMEM((K_ + 16,), jnp.uint32),
            pltpu.VMEM((K_ + 16,), jnp.int32),
            pltpu.VMEM((128,), jnp.float32),
            pltpu.VMEM((2 * K_, D_IN_), jnp.float32),
            pltpu.VMEM((128,), jnp.int32),
            pltpu.VMEM((D_IN_,), jnp.float32),
            pltpu.VMEM((D_IN_,), jnp.float32),
            pltpu.VMEM((16,), jnp.float32),
            pltpu.SemaphoreType.DMA,
            pltpu.SemaphoreType.DMA,
        ],
    )
    return f(pre2, cmax3, W_dec, b_dec)


@jax.jit
def _run(x, W_enc, b_enc, W_dec, b_dec):
    pre, cmax3 = _encode(x, W_enc, b_enc, b_dec)
    return _sc_decode(pre, cmax3, W_dec, b_dec)


def kernel(x, W_enc, b_enc, W_dec, b_dec):
    return _run(x, W_enc, b_enc, W_dec, b_dec)
